# baseline (speedup 1.0000x reference)
import jax
import jax.numpy as jnp
from jax.experimental import pallas as pl
from jax.experimental.pallas import tpu as pltpu

_N = 20000
_C = 91
_K = 100
_S = 160
_P = _N // _S
_PR = 128
_KPAD = 104
_BIG = 2**30


def _topk_kernel(x_ref, bbox_ref, scale_ref, sc_ref, lb_ref, bx_ref,
                 pm_s, pa_s):
    neg_inf = jnp.float32(-jnp.inf)

    pm_s[...] = jnp.full((_PR, _C), neg_inf, jnp.float32)
    pa_s[...] = jnp.full((_PR, _C), _BIG, jnp.int32)

    row_iota_s = jax.lax.broadcasted_iota(jnp.int32, (_S, _C), 0)

    def init_panel(p, _):
        start = p * _S
        chunk = x_ref[0, pl.ds(start, _S), :]
        pm = jnp.max(chunk, axis=0, keepdims=True)
        pa = jnp.min(jnp.where(chunk == pm, row_iota_s + start, _BIG),
                     axis=0, keepdims=True)
        pm_s[pl.ds(p, 1), :] = pm
        pa_s[pl.ds(p, 1), :] = pa
        return 0

    jax.lax.fori_loop(0, _P, init_panel, 0, unroll=2)

    sc_ref[0] = jnp.full((_KPAD, 4), neg_inf, jnp.float32)
    lb_ref[0] = jnp.zeros((_KPAD, 4), jnp.int32)
    bx_ref[0] = jnp.zeros((_KPAD, 4), jnp.float32)

    lane_c = jax.lax.broadcasted_iota(jnp.int32, (1, _C), 1)
    lane4 = jax.lax.broadcasted_iota(jnp.int32, (1, 4), 1)
    scale_row = scale_ref[0]

    def body(k, _):
        pm = pm_s[...]
        m = jnp.max(pm, axis=(0, 1), keepdims=True)
        colrow = jnp.min(jnp.where(pm == m, pa_s[...], _BIG),
                         axis=0, keepdims=True)
        flat = jnp.where(colrow < _BIG, colrow * _C + lane_c, _BIG)
        i = jnp.min(flat)
        r = i // _C
        c = i - r * _C

        sc_ref[0, pl.ds(k, 1), :] = jnp.broadcast_to(m, (1, 4))
        lb_ref[0, pl.ds(k, 1), :] = jnp.full((1, 4), c, jnp.int32)

        brow = bbox_ref[0, pl.ds(r, 1), :]
        cx = brow[:, 0:1]
        cy = brow[:, 1:2]
        w = brow[:, 2:3]
        h = brow[:, 3:4]
        box4 = jnp.where(lane4 == 0, cx - 0.5 * w,
               jnp.where(lane4 == 1, cy - 0.5 * h,
               jnp.where(lane4 == 2, cx + 0.5 * w, cy + 0.5 * h)))
        bx_ref[0, pl.ds(k, 1), :] = box4 * scale_row

        xrow = x_ref[0, pl.ds(r, 1), :]
        x_ref[0, pl.ds(r, 1), :] = jnp.where(lane_c == c, neg_inf, xrow)
        p = r // _S
        start = p * _S
        chunk = x_ref[0, pl.ds(start, _S), :]
        pmp = jnp.max(chunk, axis=0, keepdims=True)
        pap = jnp.min(jnp.where(chunk == pmp, row_iota_s + start, _BIG),
                      axis=0, keepdims=True)
        pm_s[pl.ds(p, 1), :] = pmp
        pa_s[pl.ds(p, 1), :] = pap
        return 0

    jax.lax.fori_loop(0, _K, body, 0)
    sc_ref[0] = jax.nn.sigmoid(sc_ref[0])


@jax.jit
def kernel(out_logits, out_bbox, target_sizes):
    B, N, C = out_logits.shape
    img_h = target_sizes[:, 0]
    img_w = target_sizes[:, 1]
    scale_fct = jnp.stack([img_w, img_h, img_w, img_h], axis=1)
    scale_fct = scale_fct.astype(jnp.float32).reshape(B, 1, 4)

    sc, lb, bx = pl.pallas_call(
        _topk_kernel,
        grid=(B,),
        in_specs=[
            pl.BlockSpec((1, N, C), lambda b: (b, 0, 0)),
            pl.BlockSpec((1, N, 4), lambda b: (b, 0, 0)),
            pl.BlockSpec((1, 1, 4), lambda b: (b, 0, 0)),
        ],
        out_specs=[
            pl.BlockSpec((1, _KPAD, 4), lambda b: (b, 0, 0)),
            pl.BlockSpec((1, _KPAD, 4), lambda b: (b, 0, 0)),
            pl.BlockSpec((1, _KPAD, 4), lambda b: (b, 0, 0)),
        ],
        out_shape=[
            jax.ShapeDtypeStruct((B, _KPAD, 4), jnp.float32),
            jax.ShapeDtypeStruct((B, _KPAD, 4), jnp.int32),
            jax.ShapeDtypeStruct((B, _KPAD, 4), jnp.float32),
        ],
        scratch_shapes=[
            pltpu.VMEM((_PR, _C), jnp.float32),
            pltpu.VMEM((_PR, _C), jnp.int32),
        ],
    )(out_logits, out_bbox, scale_fct)

    scores = sc[:, :_K, 0]
    labels = lb[:, :_K, 0]
    boxes = bx[:, :_K, :]
    return scores, labels, boxes


# speedup vs baseline: 1.0032x; 1.0032x over previous
import jax
import jax.numpy as jnp
from jax.experimental import pallas as pl
from jax.experimental.pallas import tpu as pltpu

_N = 20000
_C = 91
_K = 100
_S = 160
_P = _N // _S
_PR = 128
_KPAD = 104
_BIG = 2**30


def _topk_kernel(x_ref, bbox_ref, scale_ref, sc_ref, lb_ref, bx_ref,
                 pm_s, pa_s):
    neg_inf = jnp.float32(-jnp.inf)

    pm_s[...] = jnp.full((_PR, _C), neg_inf, jnp.float32)
    pa_s[...] = jnp.full((_PR, _C), _BIG, jnp.int32)

    row_iota_s = jax.lax.broadcasted_iota(jnp.int32, (_S, _C), 0)

    def init_panel(p, _):
        start = p * _S
        chunk = x_ref[0, pl.ds(start, _S), :]
        pm = jnp.max(chunk, axis=0, keepdims=True)
        pa = jnp.min(jnp.where(chunk == pm, row_iota_s + start, _BIG),
                     axis=0, keepdims=True)
        pm_s[pl.ds(p, 1), :] = pm
        pa_s[pl.ds(p, 1), :] = pa
        return 0

    jax.lax.fori_loop(0, _P, init_panel, 0, unroll=4)

    sc_ref[0] = jnp.full((_KPAD, 4), neg_inf, jnp.float32)
    lb_ref[0] = jnp.zeros((_KPAD, 4), jnp.int32)
    bx_ref[0] = jnp.zeros((_KPAD, 4), jnp.float32)

    lane_c = jax.lax.broadcasted_iota(jnp.int32, (1, _C), 1)
    lane4 = jax.lax.broadcasted_iota(jnp.int32, (1, 4), 1)
    scale_row = scale_ref[0]

    def body(k, _):
        pm = pm_s[...]
        m = jnp.max(pm, axis=(0, 1), keepdims=True)
        colrow = jnp.min(jnp.where(pm == m, pa_s[...], _BIG),
                         axis=0, keepdims=True)
        flat = jnp.where(colrow < _BIG, colrow * _C + lane_c, _BIG)
        i = jnp.min(flat)
        r = i // _C
        c = i - r * _C

        sc_ref[0, pl.ds(k, 1), :] = jnp.broadcast_to(m, (1, 4))
        lb_ref[0, pl.ds(k, 1), :] = jnp.full((1, 4), c, jnp.int32)

        brow = bbox_ref[0, pl.ds(r, 1), :]
        cx = brow[:, 0:1]
        cy = brow[:, 1:2]
        w = brow[:, 2:3]
        h = brow[:, 3:4]
        box4 = jnp.where(lane4 == 0, cx - 0.5 * w,
               jnp.where(lane4 == 1, cy - 0.5 * h,
               jnp.where(lane4 == 2, cx + 0.5 * w, cy + 0.5 * h)))
        bx_ref[0, pl.ds(k, 1), :] = box4 * scale_row

        xrow = x_ref[0, pl.ds(r, 1), :]
        x_ref[0, pl.ds(r, 1), :] = jnp.where(lane_c == c, neg_inf, xrow)
        p = r // _S
        start = p * _S
        chunk = x_ref[0, pl.ds(start, _S), :]
        pmp = jnp.max(chunk, axis=0, keepdims=True)
        pap = jnp.min(jnp.where(chunk == pmp, row_iota_s + start, _BIG),
                      axis=0, keepdims=True)
        pm_s[pl.ds(p, 1), :] = pmp
        pa_s[pl.ds(p, 1), :] = pap
        return 0

    jax.lax.fori_loop(0, _K, body, 0)
    sc_ref[0] = jax.nn.sigmoid(sc_ref[0])


@jax.jit
def kernel(out_logits, out_bbox, target_sizes):
    B, N, C = out_logits.shape
    img_h = target_sizes[:, 0]
    img_w = target_sizes[:, 1]
    scale_fct = jnp.stack([img_w, img_h, img_w, img_h], axis=1)
    scale_fct = scale_fct.astype(jnp.float32).reshape(B, 1, 4)

    sc, lb, bx = pl.pallas_call(
        _topk_kernel,
        grid=(B,),
        in_specs=[
            pl.BlockSpec((1, N, C), lambda b: (b, 0, 0)),
            pl.BlockSpec((1, N, 4), lambda b: (b, 0, 0)),
            pl.BlockSpec((1, 1, 4), lambda b: (b, 0, 0)),
        ],
        out_specs=[
            pl.BlockSpec((1, _KPAD, 4), lambda b: (b, 0, 0)),
            pl.BlockSpec((1, _KPAD, 4), lambda b: (b, 0, 0)),
            pl.BlockSpec((1, _KPAD, 4), lambda b: (b, 0, 0)),
        ],
        out_shape=[
            jax.ShapeDtypeStruct((B, _KPAD, 4), jnp.float32),
            jax.ShapeDtypeStruct((B, _KPAD, 4), jnp.int32),
            jax.ShapeDtypeStruct((B, _KPAD, 4), jnp.float32),
        ],
        scratch_shapes=[
            pltpu.VMEM((_PR, _C), jnp.float32),
            pltpu.VMEM((_PR, _C), jnp.int32),
        ],
    )(out_logits, out_bbox, scale_fct)

    scores = sc[:, :_K, 0]
    labels = lb[:, :_K, 0]
    boxes = bx[:, :_K, :]
    return scores, labels, boxes
